# Initial kernel scaffold; baseline (speedup 1.0000x reference)
import jax
import jax.numpy as jnp
from jax.experimental import pallas as pl


def kernel(features, edge_index, subgraph_adj, subgraph_norm, node_subgraph, node_list, perm, W_gcn, b_gcn, W_disc):
    raise NotImplementedError("write your pallas kernel here")


# speedup vs baseline: 16.8706x; 16.8706x over previous
import functools

import jax
import jax.numpy as jnp
from jax import lax
from jax.experimental import pallas as pl
from jax.experimental.pallas import tpu as pltpu
from jax.experimental.pallas import tpu_sc as plsc

N = 10000
E = 320000
D = 128
NS = 16
RS = 624
RC = 104
EB = 64
ECH = E // EB
CPS = 312
HB = 125
HCH = E // NS // HB

_mesh = plsc.VectorSubcoreMesh(core_axis_name="c", subcore_axis_name="s")


@functools.partial(
    pl.kernel,
    mesh=_mesh,
    out_type=[
        jax.ShapeDtypeStruct((N, D), jnp.float32),
        jax.ShapeDtypeStruct((N,), jnp.float32),
    ],
    scratch_types=[
        pltpu.VMEM((640,), jnp.int32),
        pltpu.VMEM((RC, D), jnp.float32),
        pltpu.VMEM((HCH, HB), jnp.int32),
        pltpu.VMEM((N,), jnp.float32),
        pltpu.VMEM((128,), jnp.float32),
        pltpu.VMEM_SHARED((N,), jnp.float32),
        pltpu.SemaphoreType.DMA,
    ],
)
def _sc_prep(x_hbm, perm_hbm, dsth_hbm, xp_hbm, hist_hbm,
             pidx, prow, dstbuf, histbuf, onesbuf, hist_sp, sem):
    c = lax.axis_index("c")
    s = lax.axis_index("s")
    base = s * RS

    @pl.when(c == 0)
    def _():
        pltpu.sync_copy(perm_hbm.at[pl.ds(base, RS)], pidx.at[pl.ds(0, RS)])
        for k in range(6):
            pltpu.async_copy(x_hbm.at[pidx.at[pl.ds(k * RC, RC)]],
                             prow, sem).wait()
            pltpu.sync_copy(prow, xp_hbm.at[pl.ds(base + k * RC, RC)])

        @pl.when(s == NS - 1)
        def _():
            pltpu.sync_copy(perm_hbm.at[pl.ds(15 * RS, 16)],
                            pidx.at[pl.ds(RS, 16)])
            pltpu.async_copy(x_hbm.at[pidx.at[pl.ds(RS, 16)]],
                             prow.at[pl.ds(0, 16)], sem).wait()
            pltpu.sync_copy(prow.at[pl.ds(0, 16)],
                            xp_hbm.at[pl.ds(15 * RS, 16)])

    @pl.when(jnp.logical_and(c == 1, s == 0))
    def _():
        def zero(i, carry):
            histbuf[pl.ds(i * 16, 16)] = jnp.zeros((16,), jnp.float32)
            return carry
        lax.fori_loop(0, N // 16, zero, 0)
        pltpu.sync_copy(histbuf, hist_sp)

    plsc.subcore_barrier()

    @pl.when(c == 1)
    def _():
        for i in range(8):
            onesbuf[pl.ds(i * 16, 16)] = jnp.ones((16,), jnp.float32)
        pltpu.sync_copy(dsth_hbm.at[pl.ds(s * HCH, HCH)], dstbuf)

        def body(j, carry):
            pltpu.sync_copy(onesbuf.at[pl.ds(0, HB)],
                            hist_sp.at[dstbuf.at[j]], add=True)
            return carry
        lax.fori_loop(0, HCH, body, 0)

    plsc.subcore_barrier()

    @pl.when(jnp.logical_and(c == 1, s == 0))
    def _():
        pltpu.sync_copy(hist_sp, histbuf)
        pltpu.sync_copy(histbuf, hist_hbm)


BCH = 104
NBLK = 3


@functools.partial(
    pl.kernel,
    mesh=_mesh,
    out_type=jax.ShapeDtypeStruct((2 * N, D), jnp.float32),
    scratch_types=[
        pltpu.VMEM((BCH * EB,), jnp.int32),
        pltpu.VMEM((BCH, EB), jnp.int32),
        pltpu.VMEM((EB,), jnp.int32),
        pltpu.VMEM((EB, D), jnp.float32),
        pltpu.VMEM_SHARED((N, D), jnp.float32),
        pltpu.SemaphoreType.DMA,
    ],
)
def _sc_aggregate(src_hbm, dst_hbm, tcat_hbm, out_hbm,
                  srcbuf, dstbuf, idx1d, rows, agg, sem):
    c = lax.axis_index("c")
    s = lax.axis_index("s")
    cN = c * N
    rbase = s * RS

    for k in range(9):
        pltpu.sync_copy(tcat_hbm.at[pl.ds(cN + rbase + k * 64, 64)], rows)
        pltpu.sync_copy(rows, agg.at[pl.ds(rbase + k * 64, 64)])
    pltpu.sync_copy(tcat_hbm.at[pl.ds(cN + rbase + 576, 48)],
                    rows.at[pl.ds(0, 48)])
    pltpu.sync_copy(rows.at[pl.ds(0, 48)], agg.at[pl.ds(rbase + 576, 48)])

    @pl.when(s == NS - 1)
    def _():
        pltpu.sync_copy(tcat_hbm.at[pl.ds(cN + 15 * RS, 16)],
                        rows.at[pl.ds(0, 16)])
        pltpu.sync_copy(rows.at[pl.ds(0, 16)], agg.at[pl.ds(15 * RS, 16)])

    plsc.subcore_barrier()

    cbase = s * CPS

    def chunk_body(j, carry):
        for i in range(EB // 16):
            idx1d[pl.ds(i * 16, 16)] = (
                srcbuf[pl.ds(j * EB + i * 16, 16)] + cN)
        pltpu.async_copy(tcat_hbm.at[idx1d], rows, sem).wait()
        pltpu.sync_copy(rows, agg.at[dstbuf.at[j]], add=True)
        return carry

    def do_block(c0, nch):
        pltpu.sync_copy(src_hbm.at[pl.ds(c0 * EB, nch * EB)],
                        srcbuf.at[pl.ds(0, nch * EB)])
        pltpu.sync_copy(dst_hbm.at[pl.ds(c0, nch)], dstbuf.at[pl.ds(0, nch)])
        lax.fori_loop(0, nch, chunk_body, 0)

    for t in range(NBLK):
        do_block(cbase + t * BCH, BCH)

    @pl.when(s == NS - 1)
    def _():
        do_block(cbase + NBLK * BCH, 8)

    plsc.subcore_barrier()
    for k in range(9):
        pltpu.sync_copy(agg.at[pl.ds(rbase + k * 64, 64)], rows)
        pltpu.sync_copy(rows, out_hbm.at[pl.ds(cN + rbase + k * 64, 64)])
    pltpu.sync_copy(agg.at[pl.ds(rbase + 576, 48)], rows.at[pl.ds(0, 48)])
    pltpu.sync_copy(rows.at[pl.ds(0, 48)],
                    out_hbm.at[pl.ds(cN + rbase + 576, 48)])

    @pl.when(s == NS - 1)
    def _():
        pltpu.sync_copy(agg.at[pl.ds(9984, 16)], rows.at[pl.ds(0, 16)])
        pltpu.sync_copy(rows.at[pl.ds(0, 16)],
                        out_hbm.at[pl.ds(cN + 9984, 16)])


def _tc_encode_body(x_ref, xp_ref, hist_ref, w_ref, tcat_ref, dis_ref):
    dis = lax.rsqrt(hist_ref[...] + 1.0)
    h = jnp.dot(x_ref[...], w_ref[...], preferred_element_type=jnp.float32)
    hn = jnp.dot(xp_ref[...], w_ref[...], preferred_element_type=jnp.float32)
    tcat_ref[0:N, :] = h * dis
    tcat_ref[N:2 * N, :] = hn * dis
    dis_ref[...] = dis


_tc_encode = pl.pallas_call(
    _tc_encode_body,
    out_shape=[
        jax.ShapeDtypeStruct((2 * N, D), jnp.float32),
        jax.ShapeDtypeStruct((N, 1), jnp.float32),
    ],
)


def _tc_finish_body(scat_ref, dis_ref, b_ref, wd_ref, a_ref, at_ref, norm_ref,
                    po_ref, no_ref):
    dis = dis_ref[...]
    b = b_ref[...]
    pos = jnp.maximum(scat_ref[0:N, :] * dis + b, 0.0)
    neg = jnp.maximum(scat_ref[N:2 * N, :] * dis + b, 0.0)
    m = jnp.dot(a_ref[...], pos, preferred_element_type=jnp.float32)
    ge = 1.0 / (1.0 + jnp.exp(-m / norm_ref[...]))
    summary = jnp.dot(at_ref[...], ge, preferred_element_type=jnp.float32)
    up = jnp.dot(pos, wd_ref[...], preferred_element_type=jnp.float32)
    un = jnp.dot(neg, wd_ref[...], preferred_element_type=jnp.float32)
    lp = jnp.sum(up * summary, axis=1, keepdims=True)
    ln = jnp.sum(un * summary, axis=1, keepdims=True)
    pl_loss = jnp.mean(jnp.maximum(lp, 0.0) - lp
                       + jnp.log1p(jnp.exp(-jnp.abs(lp))))
    nl_loss = jnp.mean(jnp.maximum(ln, 0.0)
                       + jnp.log1p(jnp.exp(-jnp.abs(ln))))
    po_ref[...] = jnp.reshape(pl_loss, (1, 1))
    no_ref[...] = jnp.reshape(nl_loss, (1, 1))


_tc_finish = pl.pallas_call(
    _tc_finish_body,
    out_shape=[
        jax.ShapeDtypeStruct((1, 1), jnp.float32),
        jax.ShapeDtypeStruct((1, 1), jnp.float32),
    ],
)


def kernel(features, edge_index, subgraph_adj, subgraph_norm, node_subgraph,
           node_list, perm, W_gcn, b_gcn, W_disc):
    src = edge_index[0].astype(jnp.int32)
    dst = edge_index[1].astype(jnp.int32)
    perm1d = perm.astype(jnp.int32)
    dst_h = dst.reshape(E // HB, HB)
    dst2d = dst.reshape(ECH, EB)

    xp, hist = _sc_prep(features, perm1d, dst_h)
    tcat, dis = _tc_encode(features, xp, hist.reshape(N, 1), W_gcn)
    scat = _sc_aggregate(src, dst2d, tcat)
    po, no = _tc_finish(scat, dis, b_gcn.reshape(1, D), W_disc,
                        subgraph_adj, subgraph_adj.T,
                        subgraph_norm)
    return (po[0, 0], no[0, 0])


# speedup vs baseline: 22.1892x; 1.3153x over previous
import functools

import jax
import jax.numpy as jnp
from jax import lax
from jax.experimental import pallas as pl
from jax.experimental.pallas import tpu as pltpu
from jax.experimental.pallas import tpu_sc as plsc

N = 10000
E = 320000
D = 128
NS = 16
RS = 624
RC = 104
EB = 64
ECH = E // EB
CPS = 312
HB = 125
HCH = E // NS // HB

_mesh = plsc.VectorSubcoreMesh(core_axis_name="c", subcore_axis_name="s")


@functools.partial(
    pl.kernel,
    mesh=_mesh,
    out_type=[
        jax.ShapeDtypeStruct((N, D), jnp.float32),
        jax.ShapeDtypeStruct((N,), jnp.float32),
    ],
    scratch_types=[
        pltpu.VMEM((640,), jnp.int32),
        pltpu.VMEM((RC, D), jnp.float32),
        pltpu.VMEM((HCH, HB), jnp.int32),
        pltpu.VMEM((N,), jnp.float32),
        pltpu.VMEM((128,), jnp.float32),
        pltpu.VMEM_SHARED((N,), jnp.float32),
        pltpu.SemaphoreType.DMA,
    ],
)
def _sc_prep(x_hbm, perm_hbm, dsth_hbm, xp_hbm, hist_hbm,
             pidx, prow, dstbuf, histbuf, onesbuf, hist_sp, sem):
    c = lax.axis_index("c")
    s = lax.axis_index("s")
    base = s * RS

    @pl.when(c == 0)
    def _():
        pltpu.sync_copy(perm_hbm.at[pl.ds(base, RS)], pidx.at[pl.ds(0, RS)])
        for k in range(6):
            pltpu.async_copy(x_hbm.at[pidx.at[pl.ds(k * RC, RC)]],
                             prow, sem).wait()
            pltpu.sync_copy(prow, xp_hbm.at[pl.ds(base + k * RC, RC)])

        @pl.when(s == NS - 1)
        def _():
            pltpu.sync_copy(perm_hbm.at[pl.ds(15 * RS, 16)],
                            pidx.at[pl.ds(RS, 16)])
            pltpu.async_copy(x_hbm.at[pidx.at[pl.ds(RS, 16)]],
                             prow.at[pl.ds(0, 16)], sem).wait()
            pltpu.sync_copy(prow.at[pl.ds(0, 16)],
                            xp_hbm.at[pl.ds(15 * RS, 16)])

    @pl.when(jnp.logical_and(c == 1, s == 0))
    def _():
        def zero(i, carry):
            histbuf[pl.ds(i * 16, 16)] = jnp.zeros((16,), jnp.float32)
            return carry
        lax.fori_loop(0, N // 16, zero, 0)
        pltpu.sync_copy(histbuf, hist_sp)

    plsc.subcore_barrier()

    @pl.when(c == 1)
    def _():
        for i in range(8):
            onesbuf[pl.ds(i * 16, 16)] = jnp.ones((16,), jnp.float32)
        pltpu.sync_copy(dsth_hbm.at[pl.ds(s * HCH, HCH)], dstbuf)

        def body(j, carry):
            pltpu.sync_copy(onesbuf.at[pl.ds(0, HB)],
                            hist_sp.at[dstbuf.at[j]], add=True)
            return carry
        lax.fori_loop(0, HCH, body, 0)

    plsc.subcore_barrier()

    @pl.when(jnp.logical_and(c == 1, s == 0))
    def _():
        pltpu.sync_copy(hist_sp, histbuf)
        pltpu.sync_copy(histbuf, hist_hbm)


BCH = 104
NBLK = 3


@functools.partial(
    pl.kernel,
    mesh=_mesh,
    out_type=jax.ShapeDtypeStruct((2 * N, D), jnp.float32),
    scratch_types=[
        pltpu.VMEM((BCH * EB,), jnp.int32),
        pltpu.VMEM((BCH, EB), jnp.int32),
        pltpu.VMEM((EB,), jnp.int32),
        pltpu.VMEM((EB,), jnp.int32),
        pltpu.VMEM((EB, D), jnp.float32),
        pltpu.VMEM((EB, D), jnp.float32),
        pltpu.VMEM_SHARED((N, D), jnp.float32),
        pltpu.SemaphoreType.DMA,
        pltpu.SemaphoreType.DMA,
        pltpu.SemaphoreType.DMA,
        pltpu.SemaphoreType.DMA,
    ],
)
def _sc_aggregate(src_hbm, dst_hbm, tcat_hbm, out_hbm,
                  srcbuf, dstbuf, idx0, idx1, rows0, rows1, agg,
                  gsem0, gsem1, ssem0, ssem1):
    c = lax.axis_index("c")
    s = lax.axis_index("s")
    cN = c * N
    rbase = s * RS
    idxb = (idx0, idx1)
    rowsb = (rows0, rows1)
    gsem = (gsem0, gsem1)
    ssem = (ssem0, ssem1)

    for k in range(9):
        pltpu.sync_copy(tcat_hbm.at[pl.ds(cN + rbase + k * 64, 64)], rows0)
        pltpu.sync_copy(rows0, agg.at[pl.ds(rbase + k * 64, 64)])
    pltpu.sync_copy(tcat_hbm.at[pl.ds(cN + rbase + 576, 48)],
                    rows0.at[pl.ds(0, 48)])
    pltpu.sync_copy(rows0.at[pl.ds(0, 48)], agg.at[pl.ds(rbase + 576, 48)])

    @pl.when(s == NS - 1)
    def _():
        pltpu.sync_copy(tcat_hbm.at[pl.ds(cN + 15 * RS, 16)],
                        rows0.at[pl.ds(0, 16)])
        pltpu.sync_copy(rows0.at[pl.ds(0, 16)], agg.at[pl.ds(15 * RS, 16)])

    plsc.subcore_barrier()

    cbase = s * CPS

    def prep_idx(j, b):
        for i in range(EB // 16):
            idxb[b][pl.ds(i * 16, 16)] = (
                srcbuf[pl.ds(j * EB + i * 16, 16)] + cN)

    def gather(b):
        return pltpu.make_async_copy(tcat_hbm.at[idxb[b]], rowsb[b], gsem[b])

    def scatter(j, b):
        return pltpu.make_async_copy(rowsb[b], agg.at[dstbuf.at[j]], ssem[b])

    def do_block(c0, npair):
        nch = npair * 2
        pltpu.sync_copy(src_hbm.at[pl.ds(c0 * EB, nch * EB)],
                        srcbuf.at[pl.ds(0, nch * EB)])
        pltpu.sync_copy(dst_hbm.at[pl.ds(c0, nch)], dstbuf.at[pl.ds(0, nch)])

        def pair(p, carry):
            j0 = p * 2
            for b in range(2):
                j = j0 + b

                @pl.when(p > 0)
                def _():
                    scatter(j, b).wait()
                prep_idx(j, b)
                gather(b).start()
            for b in range(2):
                j = j0 + b
                gather(b).wait()
                scatter(j, b).start(add=True)
            return carry
        lax.fori_loop(0, npair, pair, 0)
        for b in range(2):
            scatter(0, b).wait()

    for t in range(NBLK):
        do_block(cbase + t * BCH, BCH // 2)

    @pl.when(s == NS - 1)
    def _():
        do_block(cbase + NBLK * BCH, 4)

    plsc.subcore_barrier()
    for k in range(9):
        pltpu.sync_copy(agg.at[pl.ds(rbase + k * 64, 64)], rows0)
        pltpu.sync_copy(rows0, out_hbm.at[pl.ds(cN + rbase + k * 64, 64)])
    pltpu.sync_copy(agg.at[pl.ds(rbase + 576, 48)], rows0.at[pl.ds(0, 48)])
    pltpu.sync_copy(rows0.at[pl.ds(0, 48)],
                    out_hbm.at[pl.ds(cN + rbase + 576, 48)])

    @pl.when(s == NS - 1)
    def _():
        pltpu.sync_copy(agg.at[pl.ds(9984, 16)], rows0.at[pl.ds(0, 16)])
        pltpu.sync_copy(rows0.at[pl.ds(0, 16)],
                        out_hbm.at[pl.ds(cN + 9984, 16)])


def _tc_encode_body(x_ref, xp_ref, hist_ref, w_ref, tcat_ref, dis_ref):
    dis = lax.rsqrt(hist_ref[...] + 1.0)
    h = jnp.dot(x_ref[...], w_ref[...], preferred_element_type=jnp.float32)
    hn = jnp.dot(xp_ref[...], w_ref[...], preferred_element_type=jnp.float32)
    tcat_ref[0:N, :] = h * dis
    tcat_ref[N:2 * N, :] = hn * dis
    dis_ref[...] = dis


_tc_encode = pl.pallas_call(
    _tc_encode_body,
    out_shape=[
        jax.ShapeDtypeStruct((2 * N, D), jnp.float32),
        jax.ShapeDtypeStruct((N, 1), jnp.float32),
    ],
)


def _tc_finish_body(scat_ref, dis_ref, b_ref, wd_ref, a_ref, at_ref, norm_ref,
                    po_ref, no_ref):
    dis = dis_ref[...]
    b = b_ref[...]
    pos = jnp.maximum(scat_ref[0:N, :] * dis + b, 0.0)
    neg = jnp.maximum(scat_ref[N:2 * N, :] * dis + b, 0.0)
    m = jnp.dot(a_ref[...], pos, preferred_element_type=jnp.float32)
    ge = 1.0 / (1.0 + jnp.exp(-m / norm_ref[...]))
    summary = jnp.dot(at_ref[...], ge, preferred_element_type=jnp.float32)
    up = jnp.dot(pos, wd_ref[...], preferred_element_type=jnp.float32)
    un = jnp.dot(neg, wd_ref[...], preferred_element_type=jnp.float32)
    lp = jnp.sum(up * summary, axis=1, keepdims=True)
    ln = jnp.sum(un * summary, axis=1, keepdims=True)
    pl_loss = jnp.mean(jnp.maximum(lp, 0.0) - lp
                       + jnp.log1p(jnp.exp(-jnp.abs(lp))))
    nl_loss = jnp.mean(jnp.maximum(ln, 0.0)
                       + jnp.log1p(jnp.exp(-jnp.abs(ln))))
    po_ref[...] = jnp.reshape(pl_loss, (1, 1))
    no_ref[...] = jnp.reshape(nl_loss, (1, 1))


_tc_finish = pl.pallas_call(
    _tc_finish_body,
    out_shape=[
        jax.ShapeDtypeStruct((1, 1), jnp.float32),
        jax.ShapeDtypeStruct((1, 1), jnp.float32),
    ],
)


def kernel(features, edge_index, subgraph_adj, subgraph_norm, node_subgraph,
           node_list, perm, W_gcn, b_gcn, W_disc):
    src = edge_index[0].astype(jnp.int32)
    dst = edge_index[1].astype(jnp.int32)
    perm1d = perm.astype(jnp.int32)
    dst_h = dst.reshape(E // HB, HB)
    dst2d = dst.reshape(ECH, EB)

    xp, hist = _sc_prep(features, perm1d, dst_h)
    tcat, dis = _tc_encode(features, xp, hist.reshape(N, 1), W_gcn)
    scat = _sc_aggregate(src, dst2d, tcat)
    po, no = _tc_finish(scat, dis, b_gcn.reshape(1, D), W_disc,
                        subgraph_adj, subgraph_adj.T,
                        subgraph_norm)
    return (po[0, 0], no[0, 0])


# speedup vs baseline: 24.6349x; 1.1102x over previous
import functools

import jax
import jax.numpy as jnp
from jax import lax
from jax.experimental import pallas as pl
from jax.experimental.pallas import tpu as pltpu
from jax.experimental.pallas import tpu_sc as plsc

N = 10000
E = 320000
D = 128
NS = 16
RS = 624
RC = 104
EB = 128
EPAD = 327680
NDUMP = 32
ECH = EPAD // EB
CPS = ECH // NS
HB = 125
HCH = E // NS // HB

_mesh = plsc.VectorSubcoreMesh(core_axis_name="c", subcore_axis_name="s")


@functools.partial(
    pl.kernel,
    mesh=_mesh,
    out_type=[
        jax.ShapeDtypeStruct((N, D), jnp.float32),
        jax.ShapeDtypeStruct((N,), jnp.float32),
    ],
    scratch_types=[
        pltpu.VMEM((640,), jnp.int32),
        pltpu.VMEM((RC, D), jnp.float32),
        pltpu.VMEM((HCH, HB), jnp.int32),
        pltpu.VMEM((N,), jnp.float32),
        pltpu.VMEM((128,), jnp.float32),
        pltpu.VMEM_SHARED((N,), jnp.float32),
        pltpu.SemaphoreType.DMA,
    ],
)
def _sc_prep(x_hbm, perm_hbm, dsth_hbm, xp_hbm, hist_hbm,
             pidx, prow, dstbuf, histbuf, onesbuf, hist_sp, sem):
    c = lax.axis_index("c")
    s = lax.axis_index("s")
    base = s * RS

    @pl.when(c == 0)
    def _():
        pltpu.sync_copy(perm_hbm.at[pl.ds(base, RS)], pidx.at[pl.ds(0, RS)])
        for k in range(6):
            pltpu.async_copy(x_hbm.at[pidx.at[pl.ds(k * RC, RC)]],
                             prow, sem).wait()
            pltpu.sync_copy(prow, xp_hbm.at[pl.ds(base + k * RC, RC)])

        @pl.when(s == NS - 1)
        def _():
            pltpu.sync_copy(perm_hbm.at[pl.ds(15 * RS, 16)],
                            pidx.at[pl.ds(RS, 16)])
            pltpu.async_copy(x_hbm.at[pidx.at[pl.ds(RS, 16)]],
                             prow.at[pl.ds(0, 16)], sem).wait()
            pltpu.sync_copy(prow.at[pl.ds(0, 16)],
                            xp_hbm.at[pl.ds(15 * RS, 16)])

    @pl.when(jnp.logical_and(c == 1, s == 0))
    def _():
        def zero(i, carry):
            histbuf[pl.ds(i * 16, 16)] = jnp.zeros((16,), jnp.float32)
            return carry
        lax.fori_loop(0, N // 16, zero, 0)
        pltpu.sync_copy(histbuf, hist_sp)

    plsc.subcore_barrier()

    @pl.when(c == 1)
    def _():
        for i in range(8):
            onesbuf[pl.ds(i * 16, 16)] = jnp.ones((16,), jnp.float32)
        pltpu.sync_copy(dsth_hbm.at[pl.ds(s * HCH, HCH)], dstbuf)

        def body(j, carry):
            pltpu.sync_copy(onesbuf.at[pl.ds(0, HB)],
                            hist_sp.at[dstbuf.at[j]], add=True)
            return carry
        lax.fori_loop(0, HCH, body, 0)

    plsc.subcore_barrier()

    @pl.when(jnp.logical_and(c == 1, s == 0))
    def _():
        pltpu.sync_copy(hist_sp, histbuf)
        pltpu.sync_copy(histbuf, hist_hbm)


BCH = 40
NBLK = 4


@functools.partial(
    pl.kernel,
    mesh=_mesh,
    out_type=jax.ShapeDtypeStruct((2 * N, D), jnp.float32),
    scratch_types=[
        pltpu.VMEM((BCH * EB,), jnp.int32),
        pltpu.VMEM((BCH, EB), jnp.int32),
        pltpu.VMEM((EB,), jnp.int32),
        pltpu.VMEM((EB,), jnp.int32),
        pltpu.VMEM((EB, D), jnp.float32),
        pltpu.VMEM((EB, D), jnp.float32),
        pltpu.VMEM_SHARED((N + NDUMP, D), jnp.float32),
        pltpu.SemaphoreType.DMA,
        pltpu.SemaphoreType.DMA,
        pltpu.SemaphoreType.DMA,
        pltpu.SemaphoreType.DMA,
    ],
)
def _sc_aggregate(src_hbm, dst_hbm, tcat_hbm, out_hbm,
                  srcbuf, dstbuf, idx0, idx1, rows0, rows1, agg,
                  gsem0, gsem1, ssem0, ssem1):
    c = lax.axis_index("c")
    s = lax.axis_index("s")
    cN = c * N
    rbase = s * RS
    idxb = (idx0, idx1)
    rowsb = (rows0, rows1)
    gsem = (gsem0, gsem1)
    ssem = (ssem0, ssem1)

    for k in range(4):
        pltpu.sync_copy(tcat_hbm.at[pl.ds(cN + rbase + k * 128, 128)], rows0)
        pltpu.sync_copy(rows0, agg.at[pl.ds(rbase + k * 128, 128)])
    pltpu.sync_copy(tcat_hbm.at[pl.ds(cN + rbase + 512, 112)],
                    rows0.at[pl.ds(0, 112)])
    pltpu.sync_copy(rows0.at[pl.ds(0, 112)], agg.at[pl.ds(rbase + 512, 112)])

    @pl.when(s == NS - 1)
    def _():
        pltpu.sync_copy(tcat_hbm.at[pl.ds(cN + 15 * RS, 16)],
                        rows0.at[pl.ds(0, 16)])
        pltpu.sync_copy(rows0.at[pl.ds(0, 16)], agg.at[pl.ds(15 * RS, 16)])

    plsc.subcore_barrier()

    cbase = s * CPS

    def prep_idx(j, b):
        for i in range(EB // 16):
            idxb[b][pl.ds(i * 16, 16)] = (
                srcbuf[pl.ds(j * EB + i * 16, 16)] + cN)

    def gather(b):
        return pltpu.make_async_copy(tcat_hbm.at[idxb[b]], rowsb[b], gsem[b])

    def scatter(j, b):
        return pltpu.make_async_copy(rowsb[b], agg.at[dstbuf.at[j]], ssem[b])

    def do_block(c0, npair):
        nch = npair * 2
        pltpu.sync_copy(src_hbm.at[pl.ds(c0 * EB, nch * EB)],
                        srcbuf.at[pl.ds(0, nch * EB)])
        pltpu.sync_copy(dst_hbm.at[pl.ds(c0, nch)], dstbuf.at[pl.ds(0, nch)])

        def pair(p, carry):
            j0 = p * 2
            for b in range(2):
                j = j0 + b

                @pl.when(p > 0)
                def _():
                    scatter(j, b).wait()
                prep_idx(j, b)
                gather(b).start()
            for b in range(2):
                j = j0 + b
                gather(b).wait()
                scatter(j, b).start(add=True)
            return carry
        lax.fori_loop(0, npair, pair, 0)
        for b in range(2):
            scatter(0, b).wait()

    for t in range(NBLK):
        do_block(cbase + t * BCH, BCH // 2)

    plsc.subcore_barrier()
    for k in range(4):
        pltpu.sync_copy(agg.at[pl.ds(rbase + k * 128, 128)], rows0)
        pltpu.sync_copy(rows0, out_hbm.at[pl.ds(cN + rbase + k * 128, 128)])
    pltpu.sync_copy(agg.at[pl.ds(rbase + 512, 112)], rows0.at[pl.ds(0, 112)])
    pltpu.sync_copy(rows0.at[pl.ds(0, 112)],
                    out_hbm.at[pl.ds(cN + rbase + 512, 112)])

    @pl.when(s == NS - 1)
    def _():
        pltpu.sync_copy(agg.at[pl.ds(9984, 16)], rows0.at[pl.ds(0, 16)])
        pltpu.sync_copy(rows0.at[pl.ds(0, 16)],
                        out_hbm.at[pl.ds(cN + 9984, 16)])


def _tc_encode_body(x_ref, xp_ref, hist_ref, w_ref, tcat_ref, dis_ref):
    dis = lax.rsqrt(hist_ref[...] + 1.0)
    h = jnp.dot(x_ref[...], w_ref[...], preferred_element_type=jnp.float32)
    hn = jnp.dot(xp_ref[...], w_ref[...], preferred_element_type=jnp.float32)
    tcat_ref[0:N, :] = h * dis
    tcat_ref[N:2 * N, :] = hn * dis
    dis_ref[...] = dis


_tc_encode = pl.pallas_call(
    _tc_encode_body,
    out_shape=[
        jax.ShapeDtypeStruct((2 * N, D), jnp.float32),
        jax.ShapeDtypeStruct((N, 1), jnp.float32),
    ],
)


def _tc_finish_body(scat_ref, dis_ref, b_ref, wd_ref, a_ref, at_ref, norm_ref,
                    po_ref, no_ref):
    dis = dis_ref[...]
    b = b_ref[...]
    pos = jnp.maximum(scat_ref[0:N, :] * dis + b, 0.0)
    neg = jnp.maximum(scat_ref[N:2 * N, :] * dis + b, 0.0)
    m = jnp.dot(a_ref[...], pos, preferred_element_type=jnp.float32)
    ge = 1.0 / (1.0 + jnp.exp(-m / norm_ref[...]))
    summary = jnp.dot(at_ref[...], ge, preferred_element_type=jnp.float32)
    up = jnp.dot(pos, wd_ref[...], preferred_element_type=jnp.float32)
    un = jnp.dot(neg, wd_ref[...], preferred_element_type=jnp.float32)
    lp = jnp.sum(up * summary, axis=1, keepdims=True)
    ln = jnp.sum(un * summary, axis=1, keepdims=True)
    pl_loss = jnp.mean(jnp.maximum(lp, 0.0) - lp
                       + jnp.log1p(jnp.exp(-jnp.abs(lp))))
    nl_loss = jnp.mean(jnp.maximum(ln, 0.0)
                       + jnp.log1p(jnp.exp(-jnp.abs(ln))))
    po_ref[...] = jnp.reshape(pl_loss, (1, 1))
    no_ref[...] = jnp.reshape(nl_loss, (1, 1))


_tc_finish = pl.pallas_call(
    _tc_finish_body,
    out_shape=[
        jax.ShapeDtypeStruct((1, 1), jnp.float32),
        jax.ShapeDtypeStruct((1, 1), jnp.float32),
    ],
)


def kernel(features, edge_index, subgraph_adj, subgraph_norm, node_subgraph,
           node_list, perm, W_gcn, b_gcn, W_disc):
    src = edge_index[0].astype(jnp.int32)
    dst = edge_index[1].astype(jnp.int32)
    perm1d = perm.astype(jnp.int32)
    dst_h = dst.reshape(E // HB, HB)
    npad = EPAD - E
    fill = jnp.arange(npad, dtype=jnp.int32)
    src_p = jnp.concatenate([src, fill % 512])
    dst_p = jnp.concatenate([dst, N + (fill % NDUMP)])
    dst2d = dst_p.reshape(ECH, EB)

    xp, hist = _sc_prep(features, perm1d, dst_h)
    tcat, dis = _tc_encode(features, xp, hist.reshape(N, 1), W_gcn)
    scat = _sc_aggregate(src_p, dst2d, tcat)
    po, no = _tc_finish(scat, dis, b_gcn.reshape(1, D), W_disc,
                        subgraph_adj, subgraph_adj.T,
                        subgraph_norm)
    return (po[0, 0], no[0, 0])


# speedup vs baseline: 28.5562x; 1.1592x over previous
import functools

import jax
import jax.numpy as jnp
from jax import lax
from jax.experimental import pallas as pl
from jax.experimental.pallas import tpu as pltpu
from jax.experimental.pallas import tpu_sc as plsc

N = 10000
E = 320000
D = 128
NS = 16
RS = 624
RC = 104
EB = 64
NBUF = 4
EPAD = 327680
NDUMP = 32
ECH = EPAD // EB
CPS = ECH // NS
HB = 125
HCH = E // NS // HB

_mesh = plsc.VectorSubcoreMesh(core_axis_name="c", subcore_axis_name="s")


@functools.partial(
    pl.kernel,
    mesh=_mesh,
    out_type=[
        jax.ShapeDtypeStruct((N, D), jnp.float32),
        jax.ShapeDtypeStruct((N,), jnp.float32),
    ],
    scratch_types=[
        pltpu.VMEM((640,), jnp.int32),
        pltpu.VMEM((RC, D), jnp.float32),
        pltpu.VMEM((HCH, HB), jnp.int32),
        pltpu.VMEM((N,), jnp.float32),
        pltpu.VMEM((128,), jnp.float32),
        pltpu.VMEM_SHARED((N,), jnp.float32),
        pltpu.SemaphoreType.DMA,
    ],
)
def _sc_prep(x_hbm, perm_hbm, dsth_hbm, xp_hbm, hist_hbm,
             pidx, prow, dstbuf, histbuf, onesbuf, hist_sp, sem):
    c = lax.axis_index("c")
    s = lax.axis_index("s")
    base = s * RS

    @pl.when(c == 0)
    def _():
        pltpu.sync_copy(perm_hbm.at[pl.ds(base, RS)], pidx.at[pl.ds(0, RS)])
        for k in range(6):
            pltpu.async_copy(x_hbm.at[pidx.at[pl.ds(k * RC, RC)]],
                             prow, sem).wait()
            pltpu.sync_copy(prow, xp_hbm.at[pl.ds(base + k * RC, RC)])

        @pl.when(s == NS - 1)
        def _():
            pltpu.sync_copy(perm_hbm.at[pl.ds(15 * RS, 16)],
                            pidx.at[pl.ds(RS, 16)])
            pltpu.async_copy(x_hbm.at[pidx.at[pl.ds(RS, 16)]],
                             prow.at[pl.ds(0, 16)], sem).wait()
            pltpu.sync_copy(prow.at[pl.ds(0, 16)],
                            xp_hbm.at[pl.ds(15 * RS, 16)])

    @pl.when(jnp.logical_and(c == 1, s == 0))
    def _():
        def zero(i, carry):
            histbuf[pl.ds(i * 16, 16)] = jnp.zeros((16,), jnp.float32)
            return carry
        lax.fori_loop(0, N // 16, zero, 0)
        pltpu.sync_copy(histbuf, hist_sp)

    plsc.subcore_barrier()

    @pl.when(c == 1)
    def _():
        for i in range(8):
            onesbuf[pl.ds(i * 16, 16)] = jnp.ones((16,), jnp.float32)
        pltpu.sync_copy(dsth_hbm.at[pl.ds(s * HCH, HCH)], dstbuf)

        def body(j, carry):
            pltpu.sync_copy(onesbuf.at[pl.ds(0, HB)],
                            hist_sp.at[dstbuf.at[j]], add=True)
            return carry
        lax.fori_loop(0, HCH, body, 0)

    plsc.subcore_barrier()

    @pl.when(jnp.logical_and(c == 1, s == 0))
    def _():
        pltpu.sync_copy(hist_sp, histbuf)
        pltpu.sync_copy(histbuf, hist_hbm)


BCH = 64
NBLK = 5


@functools.partial(
    pl.kernel,
    mesh=_mesh,
    out_type=jax.ShapeDtypeStruct((2 * N, D), jnp.float32),
    scratch_types=[
        pltpu.VMEM((BCH * EB,), jnp.int32),
        pltpu.VMEM((BCH, EB), jnp.int32),
        pltpu.VMEM((EB,), jnp.int32),
        pltpu.VMEM((EB,), jnp.int32),
        pltpu.VMEM((EB,), jnp.int32),
        pltpu.VMEM((EB,), jnp.int32),
        pltpu.VMEM((EB, D), jnp.float32),
        pltpu.VMEM((EB, D), jnp.float32),
        pltpu.VMEM((EB, D), jnp.float32),
        pltpu.VMEM((EB, D), jnp.float32),
        pltpu.VMEM_SHARED((N + NDUMP, D), jnp.float32),
        pltpu.SemaphoreType.DMA,
        pltpu.SemaphoreType.DMA,
        pltpu.SemaphoreType.DMA,
        pltpu.SemaphoreType.DMA,
        pltpu.SemaphoreType.DMA,
        pltpu.SemaphoreType.DMA,
        pltpu.SemaphoreType.DMA,
        pltpu.SemaphoreType.DMA,
    ],
)
def _sc_aggregate(src_hbm, dst_hbm, tcat_hbm, out_hbm,
                  srcbuf, dstbuf, idx0, idx1, idx2, idx3,
                  rows0, rows1, rows2, rows3, agg,
                  gsem0, gsem1, gsem2, gsem3,
                  ssem0, ssem1, ssem2, ssem3):
    c = lax.axis_index("c")
    s = lax.axis_index("s")
    cN = c * N
    rbase = s * RS
    idxb = (idx0, idx1, idx2, idx3)
    rowsb = (rows0, rows1, rows2, rows3)
    gsem = (gsem0, gsem1, gsem2, gsem3)
    ssem = (ssem0, ssem1, ssem2, ssem3)

    for k in range(9):
        pltpu.sync_copy(tcat_hbm.at[pl.ds(cN + rbase + k * 64, 64)], rows0)
        pltpu.sync_copy(rows0, agg.at[pl.ds(rbase + k * 64, 64)])
    pltpu.sync_copy(tcat_hbm.at[pl.ds(cN + rbase + 576, 48)],
                    rows0.at[pl.ds(0, 48)])
    pltpu.sync_copy(rows0.at[pl.ds(0, 48)], agg.at[pl.ds(rbase + 576, 48)])

    @pl.when(s == NS - 1)
    def _():
        pltpu.sync_copy(tcat_hbm.at[pl.ds(cN + 15 * RS, 16)],
                        rows0.at[pl.ds(0, 16)])
        pltpu.sync_copy(rows0.at[pl.ds(0, 16)], agg.at[pl.ds(15 * RS, 16)])

    plsc.subcore_barrier()

    cbase = s * CPS

    def prep_idx(j, b):
        for i in range(EB // 16):
            idxb[b][pl.ds(i * 16, 16)] = (
                srcbuf[pl.ds(j * EB + i * 16, 16)] + cN)

    def gather(b):
        return pltpu.make_async_copy(tcat_hbm.at[idxb[b]], rowsb[b], gsem[b])

    def scatter(j, b):
        return pltpu.make_async_copy(rowsb[b], agg.at[dstbuf.at[j]], ssem[b])

    def do_block(c0, nquad):
        nch = nquad * NBUF
        pltpu.sync_copy(src_hbm.at[pl.ds(c0 * EB, nch * EB)],
                        srcbuf.at[pl.ds(0, nch * EB)])
        pltpu.sync_copy(dst_hbm.at[pl.ds(c0, nch)], dstbuf.at[pl.ds(0, nch)])

        def quad(p, carry):
            j0 = p * NBUF
            for b in range(NBUF):
                j = j0 + b

                @pl.when(p > 0)
                def _():
                    scatter(j, b).wait()
                prep_idx(j, b)
                gather(b).start()
            for b in range(NBUF):
                j = j0 + b
                gather(b).wait()
                scatter(j, b).start(add=True)
            return carry
        lax.fori_loop(0, nquad, quad, 0)
        for b in range(NBUF):
            scatter(0, b).wait()

    for t in range(NBLK):
        do_block(cbase + t * BCH, BCH // NBUF)

    plsc.subcore_barrier()
    for k in range(9):
        pltpu.sync_copy(agg.at[pl.ds(rbase + k * 64, 64)], rows0)
        pltpu.sync_copy(rows0, out_hbm.at[pl.ds(cN + rbase + k * 64, 64)])
    pltpu.sync_copy(agg.at[pl.ds(rbase + 576, 48)], rows0.at[pl.ds(0, 48)])
    pltpu.sync_copy(rows0.at[pl.ds(0, 48)],
                    out_hbm.at[pl.ds(cN + rbase + 576, 48)])

    @pl.when(s == NS - 1)
    def _():
        pltpu.sync_copy(agg.at[pl.ds(9984, 16)], rows0.at[pl.ds(0, 16)])
        pltpu.sync_copy(rows0.at[pl.ds(0, 16)],
                        out_hbm.at[pl.ds(cN + 9984, 16)])


def _tc_encode_body(x_ref, xp_ref, hist_ref, w_ref, tcat_ref, dis_ref):
    dis = lax.rsqrt(hist_ref[...] + 1.0)
    h = jnp.dot(x_ref[...], w_ref[...], preferred_element_type=jnp.float32)
    hn = jnp.dot(xp_ref[...], w_ref[...], preferred_element_type=jnp.float32)
    tcat_ref[0:N, :] = h * dis
    tcat_ref[N:2 * N, :] = hn * dis
    dis_ref[...] = dis


_tc_encode = pl.pallas_call(
    _tc_encode_body,
    out_shape=[
        jax.ShapeDtypeStruct((2 * N, D), jnp.float32),
        jax.ShapeDtypeStruct((N, 1), jnp.float32),
    ],
)


def _tc_finish_body(scat_ref, dis_ref, b_ref, wd_ref, a_ref, at_ref, norm_ref,
                    po_ref, no_ref):
    dis = dis_ref[...]
    b = b_ref[...]
    pos = jnp.maximum(scat_ref[0:N, :] * dis + b, 0.0)
    neg = jnp.maximum(scat_ref[N:2 * N, :] * dis + b, 0.0)
    m = jnp.dot(a_ref[...], pos, preferred_element_type=jnp.float32)
    ge = 1.0 / (1.0 + jnp.exp(-m / norm_ref[...]))
    summary = jnp.dot(at_ref[...], ge, preferred_element_type=jnp.float32)
    up = jnp.dot(pos, wd_ref[...], preferred_element_type=jnp.float32)
    un = jnp.dot(neg, wd_ref[...], preferred_element_type=jnp.float32)
    lp = jnp.sum(up * summary, axis=1, keepdims=True)
    ln = jnp.sum(un * summary, axis=1, keepdims=True)
    pl_loss = jnp.mean(jnp.maximum(lp, 0.0) - lp
                       + jnp.log1p(jnp.exp(-jnp.abs(lp))))
    nl_loss = jnp.mean(jnp.maximum(ln, 0.0)
                       + jnp.log1p(jnp.exp(-jnp.abs(ln))))
    po_ref[...] = jnp.reshape(pl_loss, (1, 1))
    no_ref[...] = jnp.reshape(nl_loss, (1, 1))


_tc_finish = pl.pallas_call(
    _tc_finish_body,
    out_shape=[
        jax.ShapeDtypeStruct((1, 1), jnp.float32),
        jax.ShapeDtypeStruct((1, 1), jnp.float32),
    ],
)


def kernel(features, edge_index, subgraph_adj, subgraph_norm, node_subgraph,
           node_list, perm, W_gcn, b_gcn, W_disc):
    src = edge_index[0].astype(jnp.int32)
    dst = edge_index[1].astype(jnp.int32)
    perm1d = perm.astype(jnp.int32)
    dst_h = dst.reshape(E // HB, HB)
    npad = EPAD - E
    fill = jnp.arange(npad, dtype=jnp.int32)
    src_p = jnp.concatenate([src, fill % 512])
    dst_p = jnp.concatenate([dst, N + (fill % NDUMP)])
    dst2d = dst_p.reshape(ECH, EB)

    xp, hist = _sc_prep(features, perm1d, dst_h)
    tcat, dis = _tc_encode(features, xp, hist.reshape(N, 1), W_gcn)
    scat = _sc_aggregate(src_p, dst2d, tcat)
    po, no = _tc_finish(scat, dis, b_gcn.reshape(1, D), W_disc,
                        subgraph_adj, subgraph_adj.T,
                        subgraph_norm)
    return (po[0, 0], no[0, 0])


# speedup vs baseline: 29.4619x; 1.0317x over previous
import functools

import jax
import jax.numpy as jnp
from jax import lax
from jax.experimental import pallas as pl
from jax.experimental.pallas import tpu as pltpu
from jax.experimental.pallas import tpu_sc as plsc

N = 10000
E = 320000
D = 128
NS = 16
RS = 624
RC = 104
EB = 64
NBUF = 4
EPAD = 327680
NDUMP = 32
ECH = EPAD // EB
CPS = ECH // NS
HB = 125
HCH = E // NS // HB

_mesh = plsc.VectorSubcoreMesh(core_axis_name="c", subcore_axis_name="s")


@functools.partial(
    pl.kernel,
    mesh=_mesh,
    out_type=[
        jax.ShapeDtypeStruct((N, D), jnp.float32),
        jax.ShapeDtypeStruct((N,), jnp.float32),
    ],
    scratch_types=[
        pltpu.VMEM((640,), jnp.int32),
        pltpu.VMEM((RC, D), jnp.float32),
        pltpu.VMEM((RC, D), jnp.float32),
        pltpu.VMEM((HCH, HB), jnp.int32),
        pltpu.VMEM((N,), jnp.float32),
        pltpu.VMEM((128,), jnp.float32),
        pltpu.VMEM_SHARED((N,), jnp.float32),
        pltpu.SemaphoreType.DMA,
        pltpu.SemaphoreType.DMA,
        pltpu.SemaphoreType.DMA,
    ],
)
def _sc_prep(x_hbm, perm_hbm, dsth_hbm, xp_hbm, hist_hbm,
             pidx, prow0, prow1, dstbuf, histbuf, onesbuf, hist_sp,
             psem0, psem1, hsem):
    c = lax.axis_index("c")
    s = lax.axis_index("s")
    base = s * RS
    prowb = (prow0, prow1)
    psemb = (psem0, psem1)

    @pl.when(c == 0)
    def _():
        pltpu.sync_copy(perm_hbm.at[pl.ds(base, RS)], pidx.at[pl.ds(0, RS)])

        for k in range(6):
            pltpu.async_copy(x_hbm.at[pidx.at[pl.ds(k * RC, RC)]],
                             prow0, psem0).wait()
            pltpu.sync_copy(prow0, xp_hbm.at[pl.ds(base + k * RC, RC)])

        @pl.when(s == NS - 1)
        def _():
            pltpu.sync_copy(perm_hbm.at[pl.ds(15 * RS, 16)],
                            pidx.at[pl.ds(RS, 16)])
            pltpu.async_copy(x_hbm.at[pidx.at[pl.ds(RS, 16)]],
                             prow0.at[pl.ds(0, 16)], psem0).wait()
            pltpu.sync_copy(prow0.at[pl.ds(0, 16)],
                            xp_hbm.at[pl.ds(15 * RS, 16)])

    @pl.when(jnp.logical_and(c == 1, s == 0))
    def _():
        def zero(i, carry):
            histbuf[pl.ds(i * 16, 16)] = jnp.zeros((16,), jnp.float32)
            return carry
        lax.fori_loop(0, N // 16, zero, 0)
        pltpu.sync_copy(histbuf, hist_sp)

    plsc.subcore_barrier()

    @pl.when(c == 1)
    def _():
        for i in range(8):
            onesbuf[pl.ds(i * 16, 16)] = jnp.ones((16,), jnp.float32)
        pltpu.sync_copy(dsth_hbm.at[pl.ds(s * HCH, HCH)], dstbuf)

        def group(q, carry):
            j0 = q * 16
            for b in range(16):
                pltpu.async_copy(onesbuf.at[pl.ds(0, HB)],
                                 hist_sp.at[dstbuf.at[j0 + b]], hsem,
                                 add=True)
            for b in range(16):
                pltpu.make_async_copy(onesbuf.at[pl.ds(0, HB)],
                                      hist_sp.at[dstbuf.at[j0 + b]],
                                      hsem).wait()
            return carry
        lax.fori_loop(0, HCH // 16, group, 0)

    plsc.subcore_barrier()

    @pl.when(jnp.logical_and(c == 1, s == 0))
    def _():
        pltpu.sync_copy(hist_sp, histbuf)
        pltpu.sync_copy(histbuf, hist_hbm)


BCH = 64
NBLK = 5


@functools.partial(
    pl.kernel,
    mesh=_mesh,
    out_type=jax.ShapeDtypeStruct((2 * N, D), jnp.float32),
    scratch_types=[
        pltpu.VMEM((BCH * EB,), jnp.int32),
        pltpu.VMEM((BCH, EB), jnp.int32),
        pltpu.VMEM((EB,), jnp.int32),
        pltpu.VMEM((EB,), jnp.int32),
        pltpu.VMEM((EB,), jnp.int32),
        pltpu.VMEM((EB,), jnp.int32),
        pltpu.VMEM((EB, D), jnp.float32),
        pltpu.VMEM((EB, D), jnp.float32),
        pltpu.VMEM((EB, D), jnp.float32),
        pltpu.VMEM((EB, D), jnp.float32),
        pltpu.VMEM_SHARED((N + NDUMP, D), jnp.float32),
        pltpu.SemaphoreType.DMA,
        pltpu.SemaphoreType.DMA,
        pltpu.SemaphoreType.DMA,
        pltpu.SemaphoreType.DMA,
        pltpu.SemaphoreType.DMA,
        pltpu.SemaphoreType.DMA,
        pltpu.SemaphoreType.DMA,
        pltpu.SemaphoreType.DMA,
    ],
)
def _sc_aggregate(src_hbm, dst_hbm, tcat_hbm, out_hbm,
                  srcbuf, dstbuf, idx0, idx1, idx2, idx3,
                  rows0, rows1, rows2, rows3, agg,
                  gsem0, gsem1, gsem2, gsem3,
                  ssem0, ssem1, ssem2, ssem3):
    c = lax.axis_index("c")
    s = lax.axis_index("s")
    cN = c * N
    rbase = s * RS
    idxb = (idx0, idx1, idx2, idx3)
    rowsb = (rows0, rows1, rows2, rows3)
    gsem = (gsem0, gsem1, gsem2, gsem3)
    ssem = (ssem0, ssem1, ssem2, ssem3)

    for k in range(9):
        pltpu.sync_copy(tcat_hbm.at[pl.ds(cN + rbase + k * 64, 64)], rows0)
        pltpu.sync_copy(rows0, agg.at[pl.ds(rbase + k * 64, 64)])
    pltpu.sync_copy(tcat_hbm.at[pl.ds(cN + rbase + 576, 48)],
                    rows0.at[pl.ds(0, 48)])
    pltpu.sync_copy(rows0.at[pl.ds(0, 48)], agg.at[pl.ds(rbase + 576, 48)])

    @pl.when(s == NS - 1)
    def _():
        pltpu.sync_copy(tcat_hbm.at[pl.ds(cN + 15 * RS, 16)],
                        rows0.at[pl.ds(0, 16)])
        pltpu.sync_copy(rows0.at[pl.ds(0, 16)], agg.at[pl.ds(15 * RS, 16)])

    plsc.subcore_barrier()

    cbase = s * CPS

    def prep_idx(j, b):
        for i in range(EB // 16):
            idxb[b][pl.ds(i * 16, 16)] = (
                srcbuf[pl.ds(j * EB + i * 16, 16)] + cN)

    def gather(b):
        return pltpu.make_async_copy(tcat_hbm.at[idxb[b]], rowsb[b], gsem[b])

    def scatter(j, b):
        return pltpu.make_async_copy(rowsb[b], agg.at[dstbuf.at[j]], ssem[b])

    def do_block(c0, nquad):
        nch = nquad * NBUF
        pltpu.sync_copy(src_hbm.at[pl.ds(c0 * EB, nch * EB)],
                        srcbuf.at[pl.ds(0, nch * EB)])
        pltpu.sync_copy(dst_hbm.at[pl.ds(c0, nch)], dstbuf.at[pl.ds(0, nch)])

        def quad(p, carry):
            j0 = p * NBUF
            for b in range(NBUF):
                j = j0 + b

                @pl.when(p > 0)
                def _():
                    scatter(j, b).wait()
                prep_idx(j, b)
                gather(b).start()
            for b in range(NBUF):
                j = j0 + b
                gather(b).wait()
                scatter(j, b).start(add=True)
            return carry
        lax.fori_loop(0, nquad, quad, 0)
        for b in range(NBUF):
            scatter(0, b).wait()

    for t in range(NBLK):
        do_block(cbase + t * BCH, BCH // NBUF)

    plsc.subcore_barrier()
    for k in range(9):
        pltpu.sync_copy(agg.at[pl.ds(rbase + k * 64, 64)], rows0)
        pltpu.sync_copy(rows0, out_hbm.at[pl.ds(cN + rbase + k * 64, 64)])
    pltpu.sync_copy(agg.at[pl.ds(rbase + 576, 48)], rows0.at[pl.ds(0, 48)])
    pltpu.sync_copy(rows0.at[pl.ds(0, 48)],
                    out_hbm.at[pl.ds(cN + rbase + 576, 48)])

    @pl.when(s == NS - 1)
    def _():
        pltpu.sync_copy(agg.at[pl.ds(9984, 16)], rows0.at[pl.ds(0, 16)])
        pltpu.sync_copy(rows0.at[pl.ds(0, 16)],
                        out_hbm.at[pl.ds(cN + 9984, 16)])


def _tc_encode_body(x_ref, xp_ref, hist_ref, w_ref, tcat_ref, dis_ref):
    dis = lax.rsqrt(hist_ref[...] + 1.0)
    h = jnp.dot(x_ref[...], w_ref[...], preferred_element_type=jnp.float32)
    hn = jnp.dot(xp_ref[...], w_ref[...], preferred_element_type=jnp.float32)
    tcat_ref[0:N, :] = h * dis
    tcat_ref[N:2 * N, :] = hn * dis
    dis_ref[...] = dis


_tc_encode = pl.pallas_call(
    _tc_encode_body,
    out_shape=[
        jax.ShapeDtypeStruct((2 * N, D), jnp.float32),
        jax.ShapeDtypeStruct((N, 1), jnp.float32),
    ],
)


def _tc_finish_body(scat_ref, dis_ref, b_ref, wd_ref, a_ref, at_ref, norm_ref,
                    po_ref, no_ref):
    dis = dis_ref[...]
    b = b_ref[...]
    pos = jnp.maximum(scat_ref[0:N, :] * dis + b, 0.0)
    neg = jnp.maximum(scat_ref[N:2 * N, :] * dis + b, 0.0)
    m = jnp.dot(a_ref[...], pos, preferred_element_type=jnp.float32)
    ge = 1.0 / (1.0 + jnp.exp(-m / norm_ref[...]))
    summary = jnp.dot(at_ref[...], ge, preferred_element_type=jnp.float32)
    up = jnp.dot(pos, wd_ref[...], preferred_element_type=jnp.float32)
    un = jnp.dot(neg, wd_ref[...], preferred_element_type=jnp.float32)
    lp = jnp.sum(up * summary, axis=1, keepdims=True)
    ln = jnp.sum(un * summary, axis=1, keepdims=True)
    pl_loss = jnp.mean(jnp.maximum(lp, 0.0) - lp
                       + jnp.log1p(jnp.exp(-jnp.abs(lp))))
    nl_loss = jnp.mean(jnp.maximum(ln, 0.0)
                       + jnp.log1p(jnp.exp(-jnp.abs(ln))))
    po_ref[...] = jnp.reshape(pl_loss, (1, 1))
    no_ref[...] = jnp.reshape(nl_loss, (1, 1))


_tc_finish = pl.pallas_call(
    _tc_finish_body,
    out_shape=[
        jax.ShapeDtypeStruct((1, 1), jnp.float32),
        jax.ShapeDtypeStruct((1, 1), jnp.float32),
    ],
)


def kernel(features, edge_index, subgraph_adj, subgraph_norm, node_subgraph,
           node_list, perm, W_gcn, b_gcn, W_disc):
    src = edge_index[0].astype(jnp.int32)
    dst = edge_index[1].astype(jnp.int32)
    perm1d = perm.astype(jnp.int32)
    dst_h = dst.reshape(E // HB, HB)
    npad = EPAD - E
    fill = jnp.arange(npad, dtype=jnp.int32)
    src_p = jnp.concatenate([src, fill % 512])
    dst_p = jnp.concatenate([dst, N + (fill % NDUMP)])
    dst2d = dst_p.reshape(ECH, EB)

    xp, hist = _sc_prep(features, perm1d, dst_h)
    tcat, dis = _tc_encode(features, xp, hist.reshape(N, 1), W_gcn)
    scat = _sc_aggregate(src_p, dst2d, tcat)
    po, no = _tc_finish(scat, dis, b_gcn.reshape(1, D), W_disc,
                        subgraph_adj, subgraph_adj.T,
                        subgraph_norm)
    return (po[0, 0], no[0, 0])


# speedup vs baseline: 29.5966x; 1.0046x over previous
import functools

import jax
import jax.numpy as jnp
from jax import lax
from jax.experimental import pallas as pl
from jax.experimental.pallas import tpu as pltpu
from jax.experimental.pallas import tpu_sc as plsc

N = 10000
E = 320000
D = 128
NS = 16
RS = 624
RC = 104
EB = 64
NBUF = 4
EPAD = 327680
NDUMP = 32
ECH = EPAD // EB
CPS = ECH // NS
HB = 125
HCH = E // NS // HB

_mesh = plsc.VectorSubcoreMesh(core_axis_name="c", subcore_axis_name="s")


@functools.partial(
    pl.kernel,
    mesh=_mesh,
    out_type=[
        jax.ShapeDtypeStruct((N, D), jnp.float32),
        jax.ShapeDtypeStruct((N,), jnp.float32),
    ],
    scratch_types=[
        pltpu.VMEM((640,), jnp.int32),
        pltpu.VMEM((RC, D), jnp.float32),
        pltpu.VMEM((RC, D), jnp.float32),
        pltpu.VMEM((HCH, HB), jnp.int32),
        pltpu.VMEM((N,), jnp.float32),
        pltpu.VMEM((128,), jnp.float32),
        pltpu.VMEM_SHARED((N,), jnp.float32),
        pltpu.SemaphoreType.DMA,
        pltpu.SemaphoreType.DMA,
        pltpu.SemaphoreType.DMA,
    ],
)
def _sc_prep(x_hbm, perm_hbm, dsth_hbm, xp_hbm, hist_hbm,
             pidx, prow0, prow1, dstbuf, histbuf, onesbuf, hist_sp,
             psem0, psem1, hsem):
    c = lax.axis_index("c")
    s = lax.axis_index("s")
    base = s * RS
    prowb = (prow0, prow1)
    psemb = (psem0, psem1)

    @pl.when(c == 0)
    def _():
        pltpu.sync_copy(perm_hbm.at[pl.ds(base, RS)], pidx.at[pl.ds(0, RS)])

        for k in range(6):
            pltpu.async_copy(x_hbm.at[pidx.at[pl.ds(k * RC, RC)]],
                             prow0, psem0).wait()
            pltpu.sync_copy(prow0, xp_hbm.at[pl.ds(base + k * RC, RC)])

        @pl.when(s == NS - 1)
        def _():
            pltpu.sync_copy(perm_hbm.at[pl.ds(15 * RS, 16)],
                            pidx.at[pl.ds(RS, 16)])
            pltpu.async_copy(x_hbm.at[pidx.at[pl.ds(RS, 16)]],
                             prow0.at[pl.ds(0, 16)], psem0).wait()
            pltpu.sync_copy(prow0.at[pl.ds(0, 16)],
                            xp_hbm.at[pl.ds(15 * RS, 16)])

    @pl.when(jnp.logical_and(c == 1, s == 0))
    def _():
        def zero(i, carry):
            histbuf[pl.ds(i * 16, 16)] = jnp.zeros((16,), jnp.float32)
            return carry
        lax.fori_loop(0, N // 16, zero, 0)
        pltpu.sync_copy(histbuf, hist_sp)

    plsc.subcore_barrier()

    @pl.when(c == 1)
    def _():
        for i in range(8):
            onesbuf[pl.ds(i * 16, 16)] = jnp.ones((16,), jnp.float32)
        pltpu.sync_copy(dsth_hbm.at[pl.ds(s * HCH, HCH)], dstbuf)

        def group(q, carry):
            j0 = q * 16
            for b in range(16):
                pltpu.async_copy(onesbuf.at[pl.ds(0, HB)],
                                 hist_sp.at[dstbuf.at[j0 + b]], hsem,
                                 add=True)
            for b in range(16):
                pltpu.make_async_copy(onesbuf.at[pl.ds(0, HB)],
                                      hist_sp.at[dstbuf.at[j0 + b]],
                                      hsem).wait()
            return carry
        lax.fori_loop(0, HCH // 16, group, 0)

    plsc.subcore_barrier()

    @pl.when(jnp.logical_and(c == 1, s == 0))
    def _():
        pltpu.sync_copy(hist_sp, histbuf)
        pltpu.sync_copy(histbuf, hist_hbm)


BCH = 80
NBLK = 4


@functools.partial(
    pl.kernel,
    mesh=_mesh,
    out_type=jax.ShapeDtypeStruct((2 * N, D), jnp.float32),
    scratch_types=[
        pltpu.VMEM((BCH * EB,), jnp.int32),
        pltpu.VMEM((BCH, EB), jnp.int32),
        pltpu.VMEM((EB,), jnp.int32),
        pltpu.VMEM((EB,), jnp.int32),
        pltpu.VMEM((EB,), jnp.int32),
        pltpu.VMEM((EB,), jnp.int32),
        pltpu.VMEM((EB, D), jnp.float32),
        pltpu.VMEM((EB, D), jnp.float32),
        pltpu.VMEM((EB, D), jnp.float32),
        pltpu.VMEM((EB, D), jnp.float32),
        pltpu.VMEM_SHARED((N + NDUMP, D), jnp.float32),
        pltpu.SemaphoreType.DMA,
        pltpu.SemaphoreType.DMA,
        pltpu.SemaphoreType.DMA,
        pltpu.SemaphoreType.DMA,
        pltpu.SemaphoreType.DMA,
        pltpu.SemaphoreType.DMA,
        pltpu.SemaphoreType.DMA,
        pltpu.SemaphoreType.DMA,
    ],
)
def _sc_aggregate(src_hbm, dst_hbm, tcat_hbm, out_hbm,
                  srcbuf, dstbuf, idx0, idx1, idx2, idx3,
                  rows0, rows1, rows2, rows3, agg,
                  gsem0, gsem1, gsem2, gsem3,
                  ssem0, ssem1, ssem2, ssem3):
    c = lax.axis_index("c")
    s = lax.axis_index("s")
    cN = c * N
    rbase = s * RS
    idxb = (idx0, idx1, idx2, idx3)
    rowsb = (rows0, rows1, rows2, rows3)
    gsem = (gsem0, gsem1, gsem2, gsem3)
    ssem = (ssem0, ssem1, ssem2, ssem3)

    for k in range(9):
        pltpu.sync_copy(tcat_hbm.at[pl.ds(cN + rbase + k * 64, 64)], rows0)
        pltpu.sync_copy(rows0, agg.at[pl.ds(rbase + k * 64, 64)])
    pltpu.sync_copy(tcat_hbm.at[pl.ds(cN + rbase + 576, 48)],
                    rows0.at[pl.ds(0, 48)])
    pltpu.sync_copy(rows0.at[pl.ds(0, 48)], agg.at[pl.ds(rbase + 576, 48)])

    @pl.when(s == NS - 1)
    def _():
        pltpu.sync_copy(tcat_hbm.at[pl.ds(cN + 15 * RS, 16)],
                        rows0.at[pl.ds(0, 16)])
        pltpu.sync_copy(rows0.at[pl.ds(0, 16)], agg.at[pl.ds(15 * RS, 16)])

    plsc.subcore_barrier()

    cbase = s * CPS

    def prep_idx(j, b):
        for i in range(EB // 16):
            idxb[b][pl.ds(i * 16, 16)] = (
                srcbuf[pl.ds(j * EB + i * 16, 16)] + cN)

    def gather(b):
        return pltpu.make_async_copy(tcat_hbm.at[idxb[b]], rowsb[b], gsem[b])

    def scatter(j, b):
        return pltpu.make_async_copy(rowsb[b], agg.at[dstbuf.at[j]], ssem[b])

    def do_block(c0, nquad):
        nch = nquad * NBUF
        pltpu.sync_copy(src_hbm.at[pl.ds(c0 * EB, nch * EB)],
                        srcbuf.at[pl.ds(0, nch * EB)])
        pltpu.sync_copy(dst_hbm.at[pl.ds(c0, nch)], dstbuf.at[pl.ds(0, nch)])

        def quad(p, carry):
            j0 = p * NBUF
            for b in range(NBUF):
                j = j0 + b

                @pl.when(p > 0)
                def _():
                    scatter(j, b).wait()
                prep_idx(j, b)
                gather(b).start()
            for b in range(NBUF):
                j = j0 + b
                gather(b).wait()
                scatter(j, b).start(add=True)
            return carry
        lax.fori_loop(0, nquad, quad, 0)
        for b in range(NBUF):
            scatter(0, b).wait()

    for t in range(NBLK):
        do_block(cbase + t * BCH, BCH // NBUF)

    plsc.subcore_barrier()
    for k in range(9):
        pltpu.sync_copy(agg.at[pl.ds(rbase + k * 64, 64)], rows0)
        pltpu.sync_copy(rows0, out_hbm.at[pl.ds(cN + rbase + k * 64, 64)])
    pltpu.sync_copy(agg.at[pl.ds(rbase + 576, 48)], rows0.at[pl.ds(0, 48)])
    pltpu.sync_copy(rows0.at[pl.ds(0, 48)],
                    out_hbm.at[pl.ds(cN + rbase + 576, 48)])

    @pl.when(s == NS - 1)
    def _():
        pltpu.sync_copy(agg.at[pl.ds(9984, 16)], rows0.at[pl.ds(0, 16)])
        pltpu.sync_copy(rows0.at[pl.ds(0, 16)],
                        out_hbm.at[pl.ds(cN + 9984, 16)])


def _tc_encode_body(x_ref, xp_ref, hist_ref, w_ref, tcat_ref, dis_ref):
    dis = lax.rsqrt(hist_ref[...] + 1.0)
    h = jnp.dot(x_ref[...], w_ref[...], preferred_element_type=jnp.float32)
    hn = jnp.dot(xp_ref[...], w_ref[...], preferred_element_type=jnp.float32)
    tcat_ref[0:N, :] = h * dis
    tcat_ref[N:2 * N, :] = hn * dis
    dis_ref[...] = dis


_tc_encode = pl.pallas_call(
    _tc_encode_body,
    out_shape=[
        jax.ShapeDtypeStruct((2 * N, D), jnp.float32),
        jax.ShapeDtypeStruct((N, 1), jnp.float32),
    ],
)


def _tc_finish_body(scat_ref, dis_ref, b_ref, wd_ref, a_ref, at_ref, norm_ref,
                    po_ref, no_ref):
    dis = dis_ref[...]
    b = b_ref[...]
    pos = jnp.maximum(scat_ref[0:N, :] * dis + b, 0.0)
    neg = jnp.maximum(scat_ref[N:2 * N, :] * dis + b, 0.0)
    m = jnp.dot(a_ref[...], pos, preferred_element_type=jnp.float32)
    ge = 1.0 / (1.0 + jnp.exp(-m / norm_ref[...]))
    summary = jnp.dot(at_ref[...], ge, preferred_element_type=jnp.float32)
    up = jnp.dot(pos, wd_ref[...], preferred_element_type=jnp.float32)
    un = jnp.dot(neg, wd_ref[...], preferred_element_type=jnp.float32)
    lp = jnp.sum(up * summary, axis=1, keepdims=True)
    ln = jnp.sum(un * summary, axis=1, keepdims=True)
    pl_loss = jnp.mean(jnp.maximum(lp, 0.0) - lp
                       + jnp.log1p(jnp.exp(-jnp.abs(lp))))
    nl_loss = jnp.mean(jnp.maximum(ln, 0.0)
                       + jnp.log1p(jnp.exp(-jnp.abs(ln))))
    po_ref[...] = jnp.reshape(pl_loss, (1, 1))
    no_ref[...] = jnp.reshape(nl_loss, (1, 1))


_tc_finish = pl.pallas_call(
    _tc_finish_body,
    out_shape=[
        jax.ShapeDtypeStruct((1, 1), jnp.float32),
        jax.ShapeDtypeStruct((1, 1), jnp.float32),
    ],
)


def kernel(features, edge_index, subgraph_adj, subgraph_norm, node_subgraph,
           node_list, perm, W_gcn, b_gcn, W_disc):
    src = edge_index[0].astype(jnp.int32)
    dst = edge_index[1].astype(jnp.int32)
    perm1d = perm.astype(jnp.int32)
    dst_h = dst.reshape(E // HB, HB)
    npad = EPAD - E
    fill = jnp.arange(npad, dtype=jnp.int32)
    src_p = jnp.concatenate([src, fill % 512])
    dst_p = jnp.concatenate([dst, N + (fill % NDUMP)])
    dst2d = dst_p.reshape(ECH, EB)

    xp, hist = _sc_prep(features, perm1d, dst_h)
    tcat, dis = _tc_encode(features, xp, hist.reshape(N, 1), W_gcn)
    scat = _sc_aggregate(src_p, dst2d, tcat)
    po, no = _tc_finish(scat, dis, b_gcn.reshape(1, D), W_disc,
                        subgraph_adj, subgraph_adj.T,
                        subgraph_norm)
    return (po[0, 0], no[0, 0])
